# baseline (speedup 1.0000x reference)
import functools

import jax
import jax.numpy as jnp
from jax.experimental import pallas as pl

D_MODEL_K = 4096
N_EXPERTS_K = 64
K_TOP = 8
BLOCK_T = 512
SUB_C = 256


def _gate_kernel(x_ref, w_ref, b_ref, idx_ref, nw_ref, probs_ref, mask_ref):
    logits_t = jax.lax.dot_general(
        w_ref[:], x_ref[:], (((1,), (1,)), ((), ())),
        preferred_element_type=jnp.float32,
    )
    logits_t = logits_t + b_ref[:]

    for s in range(BLOCK_T // SUB_C):
        cols = slice(s * SUB_C, (s + 1) * SUB_C)
        lt = logits_t[:, cols]
        m = jnp.max(lt, axis=0, keepdims=True)
        e = jnp.exp(lt - m)
        probs = e / jnp.sum(e, axis=0, keepdims=True)
        probs_ref[cols, :] = probs.T

        lane = jax.lax.broadcasted_iota(jnp.int32, probs.shape, 0).astype(
            jnp.float32
        )
        work = probs
        vals = []
        idxs = []
        for _ in range(K_TOP):
            mx = jnp.max(work, axis=0, keepdims=True)
            cand = jnp.where(work == mx, lane, float(N_EXPERTS_K))
            amax = jnp.min(cand, axis=0, keepdims=True)
            vals.append(mx)
            idxs.append(amax)
            work = jnp.where(lane == amax, -1.0, work)

        mask_ref[cols, :] = jnp.where(work < 0.0, 1.0, 0.0).T
        vals_cat = jnp.concatenate(vals, axis=0)
        idxs_cat = jnp.concatenate(idxs, axis=0)
        nw = vals_cat / (jnp.sum(vals_cat, axis=0, keepdims=True) + 1e-9)
        nw_ref[cols, :] = nw.T
        idx_ref[cols, :] = idxs_cat.T.astype(jnp.int32)


@jax.jit
def kernel(x, W, b):
    n_tokens = x.shape[0]
    grid = (n_tokens // BLOCK_T,)
    b2 = b.reshape(N_EXPERTS_K, 1)
    out_shapes = (
        jax.ShapeDtypeStruct((n_tokens, K_TOP), jnp.int32),
        jax.ShapeDtypeStruct((n_tokens, K_TOP), jnp.float32),
        jax.ShapeDtypeStruct((n_tokens, N_EXPERTS_K), jnp.float32),
        jax.ShapeDtypeStruct((n_tokens, N_EXPERTS_K), jnp.float32),
    )
    in_specs = [
        pl.BlockSpec((BLOCK_T, D_MODEL_K), lambda i: (i, 0)),
        pl.BlockSpec((N_EXPERTS_K, D_MODEL_K), lambda i: (0, 0)),
        pl.BlockSpec((N_EXPERTS_K, 1), lambda i: (0, 0)),
    ]
    out_specs = (
        pl.BlockSpec((BLOCK_T, K_TOP), lambda i: (i, 0)),
        pl.BlockSpec((BLOCK_T, K_TOP), lambda i: (i, 0)),
        pl.BlockSpec((BLOCK_T, N_EXPERTS_K), lambda i: (i, 0)),
        pl.BlockSpec((BLOCK_T, N_EXPERTS_K), lambda i: (i, 0)),
    )
    topk_idx, norm_weights, gate_probs, hard_mask = pl.pallas_call(
        _gate_kernel,
        grid=grid,
        in_specs=in_specs,
        out_specs=out_specs,
        out_shape=out_shapes,
    )(x, W, b2)
    return (topk_idx, norm_weights, gate_probs, hard_mask)


# speedup vs baseline: 1.0380x; 1.0380x over previous
import functools

import jax
import jax.numpy as jnp
from jax.experimental import pallas as pl

D_MODEL_K = 4096
N_EXPERTS_K = 64
K_TOP = 8
BLOCK_T = 1024
SUB_C = 256


def _gate_kernel(x_ref, w_ref, b_ref, idx_ref, nw_ref, probs_ref, mask_ref):
    logits_t = jax.lax.dot_general(
        w_ref[:], x_ref[:], (((1,), (1,)), ((), ())),
        preferred_element_type=jnp.float32,
    )
    logits_t = logits_t + b_ref[:]

    for s in range(BLOCK_T // SUB_C):
        cols = slice(s * SUB_C, (s + 1) * SUB_C)
        lt = logits_t[:, cols]
        m = jnp.max(lt, axis=0, keepdims=True)
        e = jnp.exp(lt - m)
        probs = e / jnp.sum(e, axis=0, keepdims=True)
        probs_ref[cols, :] = probs.T

        lane = jax.lax.broadcasted_iota(jnp.int32, probs.shape, 0).astype(
            jnp.float32
        )
        work = probs
        vals = []
        idxs = []
        for _ in range(K_TOP):
            mx = jnp.max(work, axis=0, keepdims=True)
            cand = jnp.where(work == mx, lane, float(N_EXPERTS_K))
            amax = jnp.min(cand, axis=0, keepdims=True)
            vals.append(mx)
            idxs.append(amax)
            work = jnp.where(lane == amax, -1.0, work)

        mask_ref[cols, :] = jnp.where(work < 0.0, 1.0, 0.0).T
        vals_cat = jnp.concatenate(vals, axis=0)
        idxs_cat = jnp.concatenate(idxs, axis=0)
        nw = vals_cat / (jnp.sum(vals_cat, axis=0, keepdims=True) + 1e-9)
        nw_ref[cols, :] = nw.T
        idx_ref[cols, :] = idxs_cat.T.astype(jnp.int32)


@jax.jit
def kernel(x, W, b):
    n_tokens = x.shape[0]
    grid = (n_tokens // BLOCK_T,)
    b2 = b.reshape(N_EXPERTS_K, 1)
    out_shapes = (
        jax.ShapeDtypeStruct((n_tokens, K_TOP), jnp.int32),
        jax.ShapeDtypeStruct((n_tokens, K_TOP), jnp.float32),
        jax.ShapeDtypeStruct((n_tokens, N_EXPERTS_K), jnp.float32),
        jax.ShapeDtypeStruct((n_tokens, N_EXPERTS_K), jnp.float32),
    )
    in_specs = [
        pl.BlockSpec((BLOCK_T, D_MODEL_K), lambda i: (i, 0)),
        pl.BlockSpec((N_EXPERTS_K, D_MODEL_K), lambda i: (0, 0)),
        pl.BlockSpec((N_EXPERTS_K, 1), lambda i: (0, 0)),
    ]
    out_specs = (
        pl.BlockSpec((BLOCK_T, K_TOP), lambda i: (i, 0)),
        pl.BlockSpec((BLOCK_T, K_TOP), lambda i: (i, 0)),
        pl.BlockSpec((BLOCK_T, N_EXPERTS_K), lambda i: (i, 0)),
        pl.BlockSpec((BLOCK_T, N_EXPERTS_K), lambda i: (i, 0)),
    )
    topk_idx, norm_weights, gate_probs, hard_mask = pl.pallas_call(
        _gate_kernel,
        grid=grid,
        in_specs=in_specs,
        out_specs=out_specs,
        out_shape=out_shapes,
    )(x, W, b2)
    return (topk_idx, norm_weights, gate_probs, hard_mask)


# speedup vs baseline: 1.0385x; 1.0005x over previous
import jax
import jax.numpy as jnp
from jax.experimental import pallas as pl

D_MODEL_K = 4096
N_EXPERTS_K = 64
K_TOP = 8
BLOCK_T = 1024
SUB_C = 256


def _gate_kernel(x_ref, w_ref, b_ref, idx_ref, nw_ref, probs_ref, mask_ref):
    logits_t = jax.lax.dot_general(
        w_ref[:], x_ref[:], (((1,), (1,)), ((), ())),
        preferred_element_type=jnp.float32,
    )
    logits_t = logits_t + b_ref[:]

    for s in range(BLOCK_T // SUB_C):
        cols = slice(s * SUB_C, (s + 1) * SUB_C)
        lt = logits_t[:, cols]
        m = jnp.max(lt, axis=0, keepdims=True)
        e = jnp.exp(lt - m)
        probs = e / jnp.sum(e, axis=0, keepdims=True)
        probs_ref[cols, :] = probs.T

        lane = jax.lax.broadcasted_iota(jnp.int32, probs.shape, 0).astype(
            jnp.float32
        )
        work = probs
        vals = []
        idxs = []
        for _ in range(K_TOP):
            mx = jnp.max(work, axis=0, keepdims=True)
            cand = jnp.where(work == mx, lane, float(N_EXPERTS_K))
            amax = jnp.min(cand, axis=0, keepdims=True)
            vals.append(mx)
            idxs.append(amax)
            work = jnp.where(lane == amax, -1.0, work)

        mask_ref[cols, :] = jnp.where(work < 0.0, 1.0, 0.0).T
        vals_cat = jnp.concatenate(vals, axis=0)
        idxs_cat = jnp.concatenate(idxs, axis=0)
        nw = vals_cat / (jnp.sum(vals_cat, axis=0, keepdims=True) + 1e-9)
        nw_ref[cols, :] = nw.T
        idx_ref[cols, :] = idxs_cat.T.astype(jnp.int32)


@jax.jit
def kernel(x, W, b):
    n_tokens = x.shape[0]
    grid = (n_tokens // BLOCK_T,)
    b2 = b.reshape(N_EXPERTS_K, 1)
    out_shapes = (
        jax.ShapeDtypeStruct((n_tokens, K_TOP), jnp.int32),
        jax.ShapeDtypeStruct((n_tokens, K_TOP), jnp.float32),
        jax.ShapeDtypeStruct((n_tokens, N_EXPERTS_K), jnp.float32),
        jax.ShapeDtypeStruct((n_tokens, N_EXPERTS_K), jnp.float32),
    )
    in_specs = [
        pl.BlockSpec((BLOCK_T, D_MODEL_K), lambda i: (i, 0)),
        pl.BlockSpec((N_EXPERTS_K, D_MODEL_K), lambda i: (0, 0)),
        pl.BlockSpec((N_EXPERTS_K, 1), lambda i: (0, 0)),
    ]
    out_specs = (
        pl.BlockSpec((BLOCK_T, K_TOP), lambda i: (i, 0)),
        pl.BlockSpec((BLOCK_T, K_TOP), lambda i: (i, 0)),
        pl.BlockSpec((BLOCK_T, N_EXPERTS_K), lambda i: (i, 0)),
        pl.BlockSpec((BLOCK_T, N_EXPERTS_K), lambda i: (i, 0)),
    )
    topk_idx, norm_weights, gate_probs, hard_mask = pl.pallas_call(
        _gate_kernel,
        grid=grid,
        in_specs=in_specs,
        out_specs=out_specs,
        out_shape=out_shapes,
    )(x, W, b2)
    return (topk_idx, norm_weights, gate_probs, hard_mask)
